# baseline (speedup 1.0000x reference)
import functools

import jax
import jax.numpy as jnp
from jax import lax
from jax.experimental import pallas as pl
from jax.experimental.pallas import tpu as pltpu
from jax.experimental.pallas import tpu_sc as plsc

N = 10000
NP = 10008
E = 320000
D = 128
NC, NS = 2, 16
NW = NC * NS
K = 128
CH = 80
NCH = NW * CH
E2 = NCH * K
RA = 632
RB = NP - (NS - 1) * RA

_mesh = plsc.VectorSubcoreMesh(core_axis_name="c", subcore_axis_name="s")


def _part_copy(src_ref, dst_ref, s, src_off, dst_off):
    r0 = pl.multiple_of(s * RA, 8)

    @pl.when(s < NS - 1)
    def _():
        pltpu.sync_copy(src_ref.at[pl.ds(pl.multiple_of(src_off + r0, 8), RA)],
                        dst_ref.at[pl.ds(pl.multiple_of(dst_off + r0, 8), RA)])

    @pl.when(s == NS - 1)
    def _():
        last = (NS - 1) * RA
        pltpu.sync_copy(src_ref.at[pl.ds(pl.multiple_of(src_off + last, 8), RB)],
                        dst_ref.at[pl.ds(pl.multiple_of(dst_off + last, 8), RB)])


@functools.partial(
    pl.kernel,
    mesh=_mesh,
    out_type=jax.ShapeDtypeStruct((NC * NP, 16), jnp.float32),
    scratch_types=[
        pltpu.VMEM((K, 16), jnp.float32),
        pltpu.VMEM((CH, K), jnp.int32),
        pltpu.VMEM_SHARED((NP, 16), jnp.float32),
    ],
)
def _deg(dst2_hbm, ones_hbm, zeros16_hbm, degw_out, ones_v, didx_v, deg_sh):
    c = lax.axis_index("c")
    s = lax.axis_index("s")
    wid = s * NC + c
    _part_copy(zeros16_hbm, deg_sh, s, 0, 0)
    pltpu.sync_copy(ones_hbm, ones_v)
    pltpu.sync_copy(dst2_hbm.at[pl.ds(wid * CH, CH)], didx_v)
    plsc.subcore_barrier()

    def body(g, carry):
        pltpu.sync_copy(ones_v, deg_sh.at[didx_v.at[g]], add=True)
        return carry

    lax.fori_loop(0, CH, body, 0)
    plsc.subcore_barrier()
    _part_copy(deg_sh, degw_out, s, 0, c * NP)


@functools.partial(
    pl.kernel,
    mesh=_mesh,
    out_type=jax.ShapeDtypeStruct((NC * NP, D), jnp.float32),
    scratch_types=[
        pltpu.VMEM((CH // 2, K), jnp.int32),
        pltpu.VMEM((CH // 2, K), jnp.int32),
        pltpu.VMEM((K, D), jnp.float32),
        pltpu.VMEM((K, D), jnp.float32),
        pltpu.VMEM_SHARED((NP, D), jnp.float32),
        pltpu.SemaphoreType.DMA,
        pltpu.SemaphoreType.DMA,
    ],
)
def _agg(y_hbm, src2_hbm, dst2_hbm, zeros_hbm, z_out,
         sidx_v, didx_v, rows0_v, rows1_v, z_sh, sem0, sem1):
    c = lax.axis_index("c")
    s = lax.axis_index("s")
    wid = s * NC + c
    cb = wid * CH
    PH = CH // 2
    _part_copy(zeros_hbm, z_sh, s, 0, 0)
    plsc.subcore_barrier()

    def fire(g, rbuf, sem):
        pltpu.async_copy(y_hbm.at[sidx_v.at[g]], rbuf, sem)

    def wait(g, rbuf, sem):
        pltpu.make_async_copy(y_hbm.at[sidx_v.at[g]], rbuf, sem).wait()

    def scat(g, rbuf):
        pltpu.sync_copy(rbuf, z_sh.at[didx_v.at[g]], add=True)

    def body(g2, carry):
        g = 2 * g2
        fire(g + 1, rows1_v, sem1)
        wait(g, rows0_v, sem0)
        scat(g, rows0_v)

        @pl.when(g2 < PH // 2 - 1)
        def _():
            fire(g + 2, rows0_v, sem0)

        wait(g + 1, rows1_v, sem1)
        scat(g + 1, rows1_v)
        return carry

    for p in range(2):
        pltpu.sync_copy(src2_hbm.at[pl.ds(cb + p * PH, PH)], sidx_v)
        pltpu.sync_copy(dst2_hbm.at[pl.ds(cb + p * PH, PH)], didx_v)
        fire(0, rows0_v, sem0)
        lax.fori_loop(0, PH // 2, body, 0)
    plsc.subcore_barrier()
    _part_copy(z_sh, z_out, s, 0, c * NP)


B = 1112
GRID = NP // B


def _pre_body(x_ref, w_ref, d0_ref, d1_ref, y_ref, dv_ref):
    deg = d0_ref[:, 0:1] + d1_ref[:, 0:1] + 1.0
    dv = jnp.broadcast_to(lax.rsqrt(deg), (B, D))
    dv_ref[...] = dv
    y_ref[...] = jnp.dot(x_ref[...], w_ref[...],
                         preferred_element_type=jnp.float32) * dv


def _mid_body(z0_ref, z1_ref, y_ref, dv_ref, b_ref, w_ref, o_ref):
    dv = dv_ref[...]
    agg = (z0_ref[...] + z1_ref[...] + y_ref[...]) * dv + b_ref[...]
    h = jnp.maximum(agg, 0.0)
    o_ref[...] = jnp.dot(h, w_ref[...], preferred_element_type=jnp.float32) * dv


def _fin_body(z0_ref, z1_ref, y_ref, dv_ref, b_ref, o_ref):
    o_ref[...] = ((z0_ref[...] + z1_ref[...] + y_ref[...]) * dv_ref[...]
                  + b_ref[...])


_row = pl.BlockSpec((B, D), lambda i: (i, 0))
_row0 = pl.BlockSpec((B, D), lambda i: (i, 0))
_row1 = pl.BlockSpec((B, D), lambda i: (i + GRID, 0))
_w = pl.BlockSpec((D, D), lambda i: (0, 0))
_bvec = pl.BlockSpec((1, D), lambda i: (0, 0))
_d0 = pl.BlockSpec((B, 16), lambda i: (i, 0))
_d1 = pl.BlockSpec((B, 16), lambda i: (i + GRID, 0))

_pre = pl.pallas_call(
    _pre_body, grid=(GRID,),
    in_specs=[_row, _w, _d0, _d1],
    out_specs=[_row, _row],
    out_shape=[jax.ShapeDtypeStruct((NP, D), jnp.float32),
               jax.ShapeDtypeStruct((NP, D), jnp.float32)],
)

_mid = pl.pallas_call(
    _mid_body, grid=(GRID,),
    in_specs=[_row0, _row1, _row, _row, _bvec, _w],
    out_specs=_row,
    out_shape=jax.ShapeDtypeStruct((NP, D), jnp.float32),
)

_fin = pl.pallas_call(
    _fin_body, grid=(GRID,),
    in_specs=[_row0, _row1, _row, _row, _bvec],
    out_specs=_row,
    out_shape=jax.ShapeDtypeStruct((NP, D), jnp.float32),
)


def kernel(x, edge_index, W0, b0, W1, b1, WF, bF):
    pad = jnp.full((E2 - E,), N, jnp.int32)
    src2 = jnp.concatenate([edge_index[0].astype(jnp.int32), pad]).reshape(NCH, K)
    dst2 = jnp.concatenate([edge_index[1].astype(jnp.int32), pad]).reshape(NCH, K)
    xp = jnp.concatenate([x, jnp.zeros((NP - N, D), jnp.float32)])
    ones16 = jnp.ones((K, 16), jnp.float32)
    zeros16 = jnp.zeros((NP, 16), jnp.float32)
    zeros = jnp.zeros((NP, D), jnp.float32)

    degw = _deg(dst2, ones16, zeros16)
    y0, dv = _pre(xp, W0, degw, degw)
    zz = _agg(y0, src2, dst2, zeros)
    y1 = _mid(zz, zz, y0, dv, b0.reshape(1, D), W1)
    zz = _agg(y1, src2, dst2, zeros)
    y2 = _mid(zz, zz, y1, dv, b1.reshape(1, D), WF)
    zz = _agg(y2, src2, dst2, zeros)
    return _fin(zz, zz, y2, dv, bF.reshape(1, D))[:N]


# speedup vs baseline: 2.7732x; 2.7732x over previous
import functools

import jax
import jax.numpy as jnp
from jax import lax
from jax.experimental import pallas as pl
from jax.experimental.pallas import tpu as pltpu
from jax.experimental.pallas import tpu_sc as plsc

N = 10000
NP = 10008
E = 320000
D = 128
NC, NS = 2, 16
NW = NC * NS
K = 128
KR = 125
CH = 80
NCH = NW * CH
E2 = NCH * K
RA = 632
RB = NP - (NS - 1) * RA

_mesh = plsc.VectorSubcoreMesh(core_axis_name="c", subcore_axis_name="s")


def _part_copy(src_ref, dst_ref, s, src_off, dst_off):
    r0 = pl.multiple_of(s * RA, 8)

    @pl.when(s < NS - 1)
    def _():
        pltpu.sync_copy(src_ref.at[pl.ds(pl.multiple_of(src_off + r0, 8), RA)],
                        dst_ref.at[pl.ds(pl.multiple_of(dst_off + r0, 8), RA)])

    @pl.when(s == NS - 1)
    def _():
        last = (NS - 1) * RA
        pltpu.sync_copy(src_ref.at[pl.ds(pl.multiple_of(src_off + last, 8), RB)],
                        dst_ref.at[pl.ds(pl.multiple_of(dst_off + last, 8), RB)])


@functools.partial(
    pl.kernel,
    mesh=_mesh,
    out_type=jax.ShapeDtypeStruct((NC * NP, 16), jnp.float32),
    scratch_types=[
        pltpu.VMEM((K, 16), jnp.float32),
        pltpu.VMEM((CH, K), jnp.int32),
        pltpu.VMEM_SHARED((NP, 16), jnp.float32),
    ],
)
def _deg(dst2_hbm, ones_hbm, zeros16_hbm, degw_out, ones_v, didx_v, deg_sh):
    c = lax.axis_index("c")
    s = lax.axis_index("s")
    wid = s * NC + c
    _part_copy(zeros16_hbm, deg_sh, s, 0, 0)
    pltpu.sync_copy(ones_hbm, ones_v)
    pltpu.sync_copy(dst2_hbm.at[pl.ds(wid * CH, CH)], didx_v)
    plsc.subcore_barrier()

    def body(g, carry):
        pltpu.sync_copy(ones_v, deg_sh.at[didx_v.at[g]], add=True)
        return carry

    lax.fori_loop(0, CH, body, 0)
    plsc.subcore_barrier()
    _part_copy(deg_sh, degw_out, s, 0, c * NP)


@functools.partial(
    pl.kernel,
    mesh=_mesh,
    out_type=jax.ShapeDtypeStruct((NC * NP, D), jnp.float32),
    scratch_types=[
        pltpu.VMEM((CH // 2, K), jnp.int32),
        pltpu.VMEM((CH // 2, K), jnp.int32),
        pltpu.VMEM((K, D), jnp.float32),
        pltpu.VMEM((K, D), jnp.float32),
        pltpu.VMEM_SHARED((NP, D), jnp.float32),
        pltpu.SemaphoreType.DMA,
        pltpu.SemaphoreType.DMA,
    ],
)
def _agg(y_hbm, src2_hbm, dst2_hbm, zeros_hbm, z_out,
         sidx_v, didx_v, rows0_v, rows1_v, z_sh, sem0, sem1):
    c = lax.axis_index("c")
    s = lax.axis_index("s")
    wid = s * NC + c
    cb = wid * CH
    PH = CH // 2
    _part_copy(zeros_hbm, z_sh, s, 0, 0)
    plsc.subcore_barrier()

    def fire(g, rbuf, sem):
        pltpu.async_copy(y_hbm.at[sidx_v.at[g]], rbuf, sem)

    def wait(g, rbuf, sem):
        pltpu.make_async_copy(y_hbm.at[sidx_v.at[g]], rbuf, sem).wait()

    def scat(g, rbuf):
        pltpu.sync_copy(rbuf, z_sh.at[didx_v.at[g]], add=True)

    def body(g2, carry):
        g = 2 * g2
        fire(g + 1, rows1_v, sem1)
        wait(g, rows0_v, sem0)
        scat(g, rows0_v)

        @pl.when(g2 < PH // 2 - 1)
        def _():
            fire(g + 2, rows0_v, sem0)

        wait(g + 1, rows1_v, sem1)
        scat(g + 1, rows1_v)
        return carry

    for p in range(2):
        pltpu.sync_copy(src2_hbm.at[pl.ds(cb + p * PH, PH)], sidx_v)
        pltpu.sync_copy(dst2_hbm.at[pl.ds(cb + p * PH, PH)], didx_v)
        fire(0, rows0_v, sem0)
        lax.fori_loop(0, PH // 2, body, 0)
    plsc.subcore_barrier()
    _part_copy(z_sh, z_out, s, 0, c * NP)


B = 1112
GRID = NP // B


def _pre_body(x_ref, w_ref, d0_ref, d1_ref, y_ref, dv_ref):
    deg = d0_ref[:, 0:1] + d1_ref[:, 0:1] + 1.0
    dv = jnp.broadcast_to(lax.rsqrt(deg), (B, D))
    dv_ref[...] = dv
    y_ref[...] = jnp.dot(x_ref[...], w_ref[...],
                         preferred_element_type=jnp.float32) * dv


def _mid_body(z0_ref, z1_ref, y_ref, dv_ref, b_ref, w_ref, o_ref):
    dv = dv_ref[...]
    agg = (z0_ref[...] + z1_ref[...] + y_ref[...]) * dv + b_ref[...]
    h = jnp.maximum(agg, 0.0)
    o_ref[...] = jnp.dot(h, w_ref[...], preferred_element_type=jnp.float32) * dv


def _fin_body(z0_ref, z1_ref, y_ref, dv_ref, b_ref, o_ref):
    o_ref[...] = ((z0_ref[...] + z1_ref[...] + y_ref[...]) * dv_ref[...]
                  + b_ref[...])


_row = pl.BlockSpec((B, D), lambda i: (i, 0))
_row0 = pl.BlockSpec((B, D), lambda i: (i, 0))
_row1 = pl.BlockSpec((B, D), lambda i: (i + GRID, 0))
_w = pl.BlockSpec((D, D), lambda i: (0, 0))
_bvec = pl.BlockSpec((1, D), lambda i: (0, 0))
_d0 = pl.BlockSpec((B, 16), lambda i: (i, 0))
_d1 = pl.BlockSpec((B, 16), lambda i: (i + GRID, 0))

_pre = pl.pallas_call(
    _pre_body, grid=(GRID,),
    in_specs=[_row, _w, _d0, _d1],
    out_specs=[_row, _row],
    out_shape=[jax.ShapeDtypeStruct((NP, D), jnp.float32),
               jax.ShapeDtypeStruct((NP, D), jnp.float32)],
)

_mid = pl.pallas_call(
    _mid_body, grid=(GRID,),
    in_specs=[_row0, _row1, _row, _row, _bvec, _w],
    out_specs=_row,
    out_shape=jax.ShapeDtypeStruct((NP, D), jnp.float32),
)

_fin = pl.pallas_call(
    _fin_body, grid=(GRID,),
    in_specs=[_row0, _row1, _row, _row, _bvec],
    out_specs=_row,
    out_shape=jax.ShapeDtypeStruct((NP, D), jnp.float32),
)


def kernel(x, edge_index, W0, b0, W1, b1, WF, bF):
    dummy = (N + (jnp.arange(NCH * (K - KR), dtype=jnp.int32)
                  % (NP - N))).reshape(NCH, K - KR)
    src2 = jnp.concatenate(
        [edge_index[0].astype(jnp.int32).reshape(NCH, KR), dummy], axis=1)
    dst2 = jnp.concatenate(
        [edge_index[1].astype(jnp.int32).reshape(NCH, KR), dummy], axis=1)
    xp = jnp.concatenate([x, jnp.zeros((NP - N, D), jnp.float32)])
    ones16 = jnp.ones((K, 16), jnp.float32)
    zeros16 = jnp.zeros((NP, 16), jnp.float32)
    zeros = jnp.zeros((NP, D), jnp.float32)

    degw = _deg(dst2, ones16, zeros16)
    y0, dv = _pre(xp, W0, degw, degw)
    zz = _agg(y0, src2, dst2, zeros)
    y1 = _mid(zz, zz, y0, dv, b0.reshape(1, D), W1)
    zz = _agg(y1, src2, dst2, zeros)
    y2 = _mid(zz, zz, y1, dv, b1.reshape(1, D), WF)
    zz = _agg(y2, src2, dst2, zeros)
    return _fin(zz, zz, y2, dv, bF.reshape(1, D))[:N]
